# Initial kernel scaffold; baseline (speedup 1.0000x reference)
import jax
import jax.numpy as jnp
from jax.experimental import pallas as pl


def kernel(x, weight):
    raise NotImplementedError("write your pallas kernel here")


# speedup vs baseline: 1.0179x; 1.0179x over previous
import jax
import jax.numpy as jnp
from jax.experimental import pallas as pl


def _copy_block(w_ref, o_ref):
    o_ref[...] = w_ref[...]


def kernel(x, weight):
    seq_len = x.shape[1]
    hidden = weight.shape[1]
    block_rows = 512
    grid = (seq_len // block_rows,)
    out = pl.pallas_call(
        _copy_block,
        grid=grid,
        in_specs=[pl.BlockSpec((block_rows, hidden), lambda i: (i, 0))],
        out_specs=pl.BlockSpec((block_rows, hidden), lambda i: (i, 0)),
        out_shape=jax.ShapeDtypeStruct((seq_len, hidden), weight.dtype),
    )(weight)
    return out[None, :, :]
